# baseline (speedup 1.0000x reference)
import functools

import jax
import jax.numpy as jnp
from jax import lax
from jax.experimental import pallas as pl
from jax.experimental.pallas import tpu as pltpu
from jax.experimental.pallas import tpu_sc as plsc

T = 2048
H = 2048
E = 64
TOPK = 8
NG = 8
GS = E // NG
TOPKG = 4
I_MOE = 1024
I_SH = 2048
SCALE = 2.5

BLK = 128
NPAD = T * TOPK + E * BLK
NB = NPAD // BLK
NBL = 256


def _first_argmax_mask(x, iota, n):
    m = jnp.max(x, axis=1, keepdims=True)
    idx = jnp.min(jnp.where(x == m, iota, n), axis=1, keepdims=True)
    return iota == idx, m


def _router_kernel(flat_ref, rw_ref, tw_ref, pos_ref, be_ref, breal_ref):
    flat = flat_ref[...]
    rw = rw_ref[...]
    logits = lax.dot_general(flat, rw, (((1,), (1,)), ((), ())),
                             preferred_element_type=jnp.float32)
    scores = jax.nn.sigmoid(logits)

    iota8 = lax.broadcasted_iota(jnp.int32, (T, GS), 1)
    grp_cols = []
    for g in range(NG):
        sub = scores[:, g * GS:(g + 1) * GS]
        sel1, m1 = _first_argmax_mask(sub, iota8, GS)
        m2 = jnp.max(jnp.where(sel1, -jnp.inf, sub), axis=1, keepdims=True)
        grp_cols.append(m1 + m2)
    grp = jnp.concatenate(grp_cols, axis=1)

    gmask = jnp.zeros((T, NG), jnp.float32)
    work = grp
    for _ in range(TOPKG):
        sel, _ = _first_argmax_mask(work, iota8, NG)
        gmask = gmask + sel.astype(jnp.float32)
        work = jnp.where(sel, -jnp.inf, work)
    smask = jnp.concatenate(
        [jnp.broadcast_to(gmask[:, g:g + 1], (T, GS)) for g in range(NG)],
        axis=1)

    masked = jnp.where(smask > 0.5, scores, 0.0)

    iota64 = lax.broadcasted_iota(jnp.int32, (T, E), 1)
    sel_list = []
    tw_cols = []
    sel8 = jnp.zeros((T, E), jnp.float32)
    work = masked
    for _ in range(TOPK):
        sel, m = _first_argmax_mask(work, iota64, E)
        sel_list.append(sel)
        tw_cols.append(m)
        sel8 = sel8 + sel.astype(jnp.float32)
        work = jnp.where(sel, -1.0, work)

    twm = jnp.concatenate(tw_cols, axis=1)
    tw = twm / (jnp.sum(twm, axis=1, keepdims=True) + 1e-20) * SCALE
    tw_ref[...] = tw

    li = lax.broadcasted_iota(jnp.int32, (BLK, BLK), 0)
    lj = lax.broadcasted_iota(jnp.int32, (BLK, BLK), 1)
    Ls = (lj < li).astype(jnp.float32)
    carry = jnp.zeros((1, E), jnp.float32)
    tp_chunks = []
    for c in range(T // BLK):
        s = sel8[c * BLK:(c + 1) * BLK, :]
        tp = lax.dot_general(Ls, s, (((1,), (0,)), ((), ())),
                             preferred_element_type=jnp.float32)
        tp_chunks.append(tp + jnp.broadcast_to(carry, (BLK, E)))
        carry = carry + jnp.sum(s, axis=0, keepdims=True)
    tokprefix = jnp.concatenate(tp_chunks, axis=0)
    counts_row = carry

    pc_row = jnp.ceil(counts_row * (1.0 / BLK)) * BLK
    mi = lax.broadcasted_iota(jnp.int32, (E, E), 0)
    mj = lax.broadcasted_iota(jnp.int32, (E, E), 1)
    Mrow = (mi < mj).astype(jnp.float32)
    po_row = lax.dot_general(pc_row, Mrow, (((1,), (0,)), ((), ())),
                             preferred_element_type=jnp.float32)

    slot_val = jnp.broadcast_to(po_row, (T, E)) + tokprefix
    pos_cols = []
    for sel in sel_list:
        pos_cols.append(jnp.sum(jnp.where(sel, slot_val, 0.0), axis=1,
                                keepdims=True))
    pos = jnp.concatenate(pos_cols, axis=1)
    pos_ref[...] = pos.astype(jnp.int32)

    ones_col = jnp.ones((T, 1), jnp.float32)
    counts_col = lax.dot_general(sel8, ones_col, (((0,), (0,)), ((), ())),
                                 preferred_element_type=jnp.float32)
    pc_col = jnp.ceil(counts_col * (1.0 / BLK)) * BLK
    Mcol = (mj < mi).astype(jnp.float32)
    po_col = lax.dot_general(Mcol, pc_col, (((1,), (0,)), ((), ())),
                             preferred_element_type=jnp.float32)

    biota = lax.broadcasted_iota(jnp.int32, (E, NBL), 1).astype(jnp.float32)
    start_blk = jnp.broadcast_to(po_col * (1.0 / BLK), (E, NBL))
    be = jnp.sum((start_blk <= biota).astype(jnp.float32), axis=0,
                 keepdims=True) - 1.0
    limit_col = po_col + counts_col
    eiota = lax.broadcasted_iota(jnp.int32, (E, NBL), 0).astype(jnp.float32)
    eq = jnp.broadcast_to(be, (E, NBL)) == eiota
    limit_b = jnp.sum(jnp.where(eq, jnp.broadcast_to(limit_col, (E, NBL)),
                                0.0), axis=0, keepdims=True)
    biota_row = lax.broadcasted_iota(jnp.int32, (1, NBL), 1).astype(jnp.float32)
    breal = (biota_row * BLK < limit_b).astype(jnp.int32)

    be_ref[...] = jnp.broadcast_to(be.astype(jnp.int32), (8, NBL))
    breal_ref[...] = jnp.broadcast_to(breal, (8, NBL))


def _router(flat, rw):
    return pl.pallas_call(
        _router_kernel,
        out_shape=(
            jax.ShapeDtypeStruct((T, TOPK), jnp.float32),
            jax.ShapeDtypeStruct((T, TOPK), jnp.int32),
            jax.ShapeDtypeStruct((8, NBL), jnp.int32),
            jax.ShapeDtypeStruct((8, NBL), jnp.int32),
        ),
    )(flat, rw)


NC = 2
NS = 16
NWK = NC * NS
NPAIR = T * TOPK
PPW = NPAIR // NS
RPW = NPAD // NWK
GA = 32
NCH = RPW // GA
PPB = NPAIR // NWK
GB = 32
NCB = PPB // GB


def _sc_build_and_gather(pos_flat, zeros_i32, flat):
    mesh = plsc.VectorSubcoreMesh(core_axis_name="c", subcore_axis_name="s")

    @functools.partial(
        pl.kernel, mesh=mesh,
        out_type=jax.ShapeDtypeStruct((NPAD, H), jnp.float32),
        scratch_types=[
            pltpu.VMEM((PPW,), jnp.int32),
            pltpu.VMEM((PPW,), jnp.int32),
            pltpu.VMEM((RPW,), jnp.int32),
            pltpu.VMEM((GA, H), jnp.float32),
            pltpu.VMEM_SHARED((NPAD,), jnp.int32),
            pltpu.SemaphoreType.DMA,
        ],
    )
    def k(pos_hbm, zeros_hbm, x_hbm, xs_hbm, idx_v, val_v, idx_all,
          rows0, rt_sh, gs0):
        cid = lax.axis_index("c")
        sid = lax.axis_index("s")

        zchunk = NPAD // NS
        pltpu.sync_copy(zeros_hbm.at[pl.ds(sid * zchunk, zchunk)],
                        rt_sh.at[pl.ds(sid * zchunk, zchunk)])
        plsc.subcore_barrier()
        base = sid * PPW
        pltpu.sync_copy(pos_hbm.at[pl.ds(base, PPW)], idx_v)

        def abody(i, carry):
            val_v[pl.ds(i * 16, 16)] = jax.lax.shift_right_logical(
                base + i * 16 + lax.broadcasted_iota(jnp.int32, (16,), 0), 3)
            return carry

        lax.fori_loop(0, PPW // 16, abody, 0)
        pltpu.sync_copy(val_v, rt_sh.at[idx_v])
        plsc.subcore_barrier()

        wid = sid * NC + cid
        rbase = wid * RPW
        pltpu.sync_copy(rt_sh.at[pl.ds(rbase, RPW)], idx_all)

        def bbody(c, carry):
            pltpu.async_copy(
                x_hbm.at[idx_all.at[pl.ds(c * GA, GA)]], rows0, gs0).wait()
            pltpu.sync_copy(rows0, xs_hbm.at[pl.ds(rbase + c * GA, GA)])
            return carry

        lax.fori_loop(0, NCH, bbody, 0)

    return k(pos_flat, zeros_i32, flat)


def _sc_gather_pairs(pos_flat, ys):
    mesh = plsc.VectorSubcoreMesh(core_axis_name="c", subcore_axis_name="s")

    @functools.partial(
        pl.kernel, mesh=mesh,
        out_type=jax.ShapeDtypeStruct((NPAIR, H), jnp.float32),
        scratch_types=[
            pltpu.VMEM((GB,), jnp.int32),
            pltpu.VMEM((GB, H), jnp.float32),
            pltpu.SemaphoreType.DMA,
        ],
    )
    def k(pos_hbm, ys_hbm, ysg_hbm, idxg, rows, sem):
        wid = lax.axis_index("s") * NC + lax.axis_index("c")
        bbase = wid * PPB

        def chunk(c, carry):
            off = bbase + c * GB
            pltpu.sync_copy(pos_hbm.at[pl.ds(off, GB)], idxg)
            pltpu.async_copy(ys_hbm.at[idxg], rows, sem).wait()
            pltpu.sync_copy(rows, ysg_hbm.at[pl.ds(off, GB)])
            return carry

        lax.fori_loop(0, NCB, chunk, 0)

    return k(pos_flat, ys)


def _gemm_kernel(be_ref, breal_ref, xs_ref, g_ref, u_ref, d_ref, ys_ref):
    blk = pl.program_id(0)

    @pl.when(breal_ref[blk] == 1)
    def _():
        x = xs_ref[...]
        g = g_ref[0]
        u = u_ref[0]
        d = d_ref[0]
        a = lax.dot_general(x, g, (((1,), (1,)), ((), ())),
                            preferred_element_type=jnp.float32)
        b = lax.dot_general(x, u, (((1,), (1,)), ((), ())),
                            preferred_element_type=jnp.float32)
        h = a * jax.nn.sigmoid(a) * b
        ys_ref[...] = lax.dot_general(h, d, (((1,), (1,)), ((), ())),
                                      preferred_element_type=jnp.float32)


def _grouped_gemm(xs, gate_w, up_w, down_w, block_expert, block_real):
    grid_spec = pltpu.PrefetchScalarGridSpec(
        num_scalar_prefetch=2,
        grid=(NB,),
        in_specs=[
            pl.BlockSpec((BLK, H), lambda b, be, br: (b, 0)),
            pl.BlockSpec((1, I_MOE, H), lambda b, be, br: (be[b], 0, 0)),
            pl.BlockSpec((1, I_MOE, H), lambda b, be, br: (be[b], 0, 0)),
            pl.BlockSpec((1, H, I_MOE), lambda b, be, br: (be[b], 0, 0)),
        ],
        out_specs=pl.BlockSpec((BLK, H), lambda b, be, br: (b, 0)),
    )
    return pl.pallas_call(
        _gemm_kernel,
        grid_spec=grid_spec,
        out_shape=jax.ShapeDtypeStruct((NPAD, H), jnp.float32),
    )(block_expert, block_real, xs, gate_w, up_w, down_w)


TT = 256
IT = 512


def _shared_kernel(x_ref, g_ref, u_ref, d_ref, o_ref):
    i = pl.program_id(1)
    x = x_ref[...]
    a = lax.dot_general(x, g_ref[...], (((1,), (1,)), ((), ())),
                        preferred_element_type=jnp.float32)
    b = lax.dot_general(x, u_ref[...], (((1,), (1,)), ((), ())),
                        preferred_element_type=jnp.float32)
    h = a * jax.nn.sigmoid(a) * b
    y = lax.dot_general(h, d_ref[...], (((1,), (1,)), ((), ())),
                        preferred_element_type=jnp.float32)

    @pl.when(i == 0)
    def _():
        o_ref[...] = y

    @pl.when(i > 0)
    def _():
        o_ref[...] = o_ref[...] + y


def _shared_mlp(flat, gw, uw, dw):
    return pl.pallas_call(
        _shared_kernel,
        grid=(T // TT, I_SH // IT),
        in_specs=[
            pl.BlockSpec((TT, H), lambda t, i: (t, 0)),
            pl.BlockSpec((IT, H), lambda t, i: (i, 0)),
            pl.BlockSpec((IT, H), lambda t, i: (i, 0)),
            pl.BlockSpec((H, IT), lambda t, i: (0, i)),
        ],
        out_specs=pl.BlockSpec((TT, H), lambda t, i: (t, 0)),
        out_shape=jax.ShapeDtypeStruct((T, H), jnp.float32),
    )(flat, gw, uw, dw)


CT = 64


def _combine_kernel(ysg_ref, tw_ref, sh_ref, o_ref):
    acc = sh_ref[...]
    tw = tw_ref[...]
    for k in range(TOPK):
        w = jnp.broadcast_to(tw[:, k:k + 1], (CT, H))
        acc = acc + w * ysg_ref[:, k, :]
    o_ref[...] = acc


def _combine(ysg3, tw, shared):
    return pl.pallas_call(
        _combine_kernel,
        grid=(T // CT,),
        in_specs=[
            pl.BlockSpec((CT, TOPK, H), lambda t: (t, 0, 0)),
            pl.BlockSpec((CT, TOPK), lambda t: (t, 0)),
            pl.BlockSpec((CT, H), lambda t: (t, 0)),
        ],
        out_specs=pl.BlockSpec((CT, H), lambda t: (t, 0)),
        out_shape=jax.ShapeDtypeStruct((T, H), jnp.float32),
    )(ysg3, tw, shared)


def kernel(hidden_states, router_weight, gate_w, up_w, down_w,
           sh_gate_w, sh_up_w, sh_down_w):
    orig = hidden_states.shape
    flat = hidden_states.reshape(T, H)

    tw, pos, be8, breal8 = _router(flat, router_weight)
    block_expert = be8[0, :NB]
    block_real = breal8[0, :NB]

    pos_flat = pos.reshape(NPAIR)
    zeros_i32 = jnp.zeros((NPAD,), jnp.int32)
    xs = _sc_build_and_gather(pos_flat, zeros_i32, flat)

    shared = _shared_mlp(flat, sh_gate_w, sh_up_w, sh_down_w)

    ys = _grouped_gemm(xs, gate_w, up_w, down_w, block_expert, block_real)

    ysg3 = _sc_gather_pairs(pos_flat, ys).reshape(T, TOPK, H)

    out = _combine(ysg3, tw, shared)
    return out.reshape(orig)


# speedup vs baseline: 1.3060x; 1.3060x over previous
import functools

import jax
import jax.numpy as jnp
from jax import lax
from jax.experimental import pallas as pl
from jax.experimental.pallas import tpu as pltpu
from jax.experimental.pallas import tpu_sc as plsc

T = 2048
H = 2048
E = 64
TOPK = 8
NG = 8
GS = E // NG
TOPKG = 4
I_MOE = 1024
I_SH = 2048
SCALE = 2.5

BLK = 128
NPAD = T * TOPK + E * BLK
NB = NPAD // BLK
NBL = 256


def _first_argmax_mask(x, iota, n):
    m = jnp.max(x, axis=1, keepdims=True)
    idx = jnp.min(jnp.where(x == m, iota, n), axis=1, keepdims=True)
    return iota == idx, m


def _router_kernel(flat_ref, rw_ref, tw_ref, pos_ref, be_ref, breal_ref):
    flat = flat_ref[...]
    rw = rw_ref[...]
    logits = lax.dot_general(flat, rw, (((1,), (1,)), ((), ())),
                             preferred_element_type=jnp.float32)
    scores = jax.nn.sigmoid(logits)

    iota8 = lax.broadcasted_iota(jnp.int32, (T, GS), 1)
    grp_cols = []
    for g in range(NG):
        sub = scores[:, g * GS:(g + 1) * GS]
        sel1, m1 = _first_argmax_mask(sub, iota8, GS)
        m2 = jnp.max(jnp.where(sel1, -jnp.inf, sub), axis=1, keepdims=True)
        grp_cols.append(m1 + m2)
    grp = jnp.concatenate(grp_cols, axis=1)

    gmask = jnp.zeros((T, NG), jnp.float32)
    work = grp
    for _ in range(TOPKG):
        sel, _ = _first_argmax_mask(work, iota8, NG)
        gmask = gmask + sel.astype(jnp.float32)
        work = jnp.where(sel, -jnp.inf, work)
    smask = jnp.concatenate(
        [jnp.broadcast_to(gmask[:, g:g + 1], (T, GS)) for g in range(NG)],
        axis=1)

    masked = jnp.where(smask > 0.5, scores, 0.0)

    iota64 = lax.broadcasted_iota(jnp.int32, (T, E), 1)
    sel_list = []
    tw_cols = []
    sel8 = jnp.zeros((T, E), jnp.float32)
    work = masked
    for _ in range(TOPK):
        sel, m = _first_argmax_mask(work, iota64, E)
        sel_list.append(sel)
        tw_cols.append(m)
        sel8 = sel8 + sel.astype(jnp.float32)
        work = jnp.where(sel, -1.0, work)

    twm = jnp.concatenate(tw_cols, axis=1)
    tw = twm / (jnp.sum(twm, axis=1, keepdims=True) + 1e-20) * SCALE
    tw_ref[...] = tw

    li = lax.broadcasted_iota(jnp.int32, (BLK, BLK), 0)
    lj = lax.broadcasted_iota(jnp.int32, (BLK, BLK), 1)
    Ls = (lj < li).astype(jnp.float32)
    carry = jnp.zeros((1, E), jnp.float32)
    tp_chunks = []
    for c in range(T // BLK):
        s = sel8[c * BLK:(c + 1) * BLK, :]
        tp = lax.dot_general(Ls, s, (((1,), (0,)), ((), ())),
                             preferred_element_type=jnp.float32)
        tp_chunks.append(tp + jnp.broadcast_to(carry, (BLK, E)))
        carry = carry + jnp.sum(s, axis=0, keepdims=True)
    tokprefix = jnp.concatenate(tp_chunks, axis=0)
    counts_row = carry

    pc_row = jnp.ceil(counts_row * (1.0 / BLK)) * BLK
    mi = lax.broadcasted_iota(jnp.int32, (E, E), 0)
    mj = lax.broadcasted_iota(jnp.int32, (E, E), 1)
    Mrow = (mi < mj).astype(jnp.float32)
    po_row = lax.dot_general(pc_row, Mrow, (((1,), (0,)), ((), ())),
                             preferred_element_type=jnp.float32)

    slot_val = jnp.broadcast_to(po_row, (T, E)) + tokprefix
    pos_cols = []
    for sel in sel_list:
        pos_cols.append(jnp.sum(jnp.where(sel, slot_val, 0.0), axis=1,
                                keepdims=True))
    pos = jnp.concatenate(pos_cols, axis=1)
    pos_ref[...] = pos.astype(jnp.int32)

    ones_col = jnp.ones((T, 1), jnp.float32)
    counts_col = lax.dot_general(sel8, ones_col, (((0,), (0,)), ((), ())),
                                 preferred_element_type=jnp.float32)
    pc_col = jnp.ceil(counts_col * (1.0 / BLK)) * BLK
    Mcol = (mj < mi).astype(jnp.float32)
    po_col = lax.dot_general(Mcol, pc_col, (((1,), (0,)), ((), ())),
                             preferred_element_type=jnp.float32)

    biota = lax.broadcasted_iota(jnp.int32, (E, NBL), 1).astype(jnp.float32)
    start_blk = jnp.broadcast_to(po_col * (1.0 / BLK), (E, NBL))
    be = jnp.sum((start_blk <= biota).astype(jnp.float32), axis=0,
                 keepdims=True) - 1.0
    limit_col = po_col + counts_col
    eiota = lax.broadcasted_iota(jnp.int32, (E, NBL), 0).astype(jnp.float32)
    eq = jnp.broadcast_to(be, (E, NBL)) == eiota
    limit_b = jnp.sum(jnp.where(eq, jnp.broadcast_to(limit_col, (E, NBL)),
                                0.0), axis=0, keepdims=True)
    biota_row = lax.broadcasted_iota(jnp.int32, (1, NBL), 1).astype(jnp.float32)
    breal = (biota_row * BLK < limit_b).astype(jnp.int32)

    be_ref[...] = jnp.broadcast_to(be.astype(jnp.int32), (8, NBL))
    breal_ref[...] = jnp.broadcast_to(breal, (8, NBL))


def _router(flat, rw):
    return pl.pallas_call(
        _router_kernel,
        out_shape=(
            jax.ShapeDtypeStruct((T, TOPK), jnp.float32),
            jax.ShapeDtypeStruct((T, TOPK), jnp.int32),
            jax.ShapeDtypeStruct((8, NBL), jnp.int32),
            jax.ShapeDtypeStruct((8, NBL), jnp.int32),
        ),
    )(flat, rw)


NC = 2
NS = 16
NWK = NC * NS
NPAIR = T * TOPK
PPW = NPAIR // NS
RPW = NPAD // NWK
GA = 32
NCH = RPW // GA
PPB = NPAIR // NWK
GB = 32
NCB = PPB // GB


def _sc_build_and_gather(pos_flat, zeros_i32, flat):
    mesh = plsc.VectorSubcoreMesh(core_axis_name="c", subcore_axis_name="s")

    @functools.partial(
        pl.kernel, mesh=mesh,
        out_type=jax.ShapeDtypeStruct((NPAD, H), jnp.float32),
        scratch_types=[
            pltpu.VMEM((PPW,), jnp.int32),
            pltpu.VMEM((PPW,), jnp.int32),
            pltpu.VMEM((GA,), jnp.int32),
            pltpu.VMEM((GA, H), jnp.float32),
            pltpu.VMEM_SHARED((NPAD,), jnp.int32),
            pltpu.SemaphoreType.DMA,
        ],
    )
    def k(pos_hbm, zeros_hbm, x_hbm, xs_hbm, idx_v, val_v, idx_all,
          rows0, rt_sh, gs0):
        cid = lax.axis_index("c")
        sid = lax.axis_index("s")

        zchunk = NPAD // NS
        pltpu.sync_copy(zeros_hbm.at[pl.ds(sid * zchunk, zchunk)],
                        rt_sh.at[pl.ds(sid * zchunk, zchunk)])
        plsc.subcore_barrier()
        base = sid * PPW
        pltpu.sync_copy(pos_hbm.at[pl.ds(base, PPW)], idx_v)

        def abody(i, carry):
            val_v[pl.ds(i * 16, 16)] = jax.lax.shift_right_logical(
                base + i * 16 + lax.broadcasted_iota(jnp.int32, (16,), 0), 3)
            return carry

        lax.fori_loop(0, PPW // 16, abody, 0)
        pltpu.sync_copy(val_v, rt_sh.at[idx_v])
        plsc.subcore_barrier()

        wid = sid * NC + cid
        rbase = wid * RPW

        def bbody(c, carry):
            off = rbase + c * GA
            pltpu.sync_copy(rt_sh.at[pl.ds(off, GA)], idx_all)
            pltpu.async_copy(x_hbm.at[idx_all], rows0, gs0).wait()
            pltpu.sync_copy(rows0, xs_hbm.at[pl.ds(off, GA)])
            return carry

        lax.fori_loop(0, NCH, bbody, 0)

    return k(pos_flat, zeros_i32, flat)


def _sc_gather_pairs(pos_flat, ys):
    mesh = plsc.VectorSubcoreMesh(core_axis_name="c", subcore_axis_name="s")

    @functools.partial(
        pl.kernel, mesh=mesh,
        out_type=jax.ShapeDtypeStruct((NPAIR, H), jnp.float32),
        scratch_types=[
            pltpu.VMEM((GB,), jnp.int32),
            pltpu.VMEM((GB, H), jnp.float32),
            pltpu.SemaphoreType.DMA,
        ],
    )
    def k(pos_hbm, ys_hbm, ysg_hbm, idxg, rows, sem):
        wid = lax.axis_index("s") * NC + lax.axis_index("c")
        bbase = wid * PPB

        def chunk(c, carry):
            off = bbase + c * GB
            pltpu.sync_copy(pos_hbm.at[pl.ds(off, GB)], idxg)
            pltpu.async_copy(ys_hbm.at[idxg], rows, sem).wait()
            pltpu.sync_copy(rows, ysg_hbm.at[pl.ds(off, GB)])
            return carry

        lax.fori_loop(0, NCB, chunk, 0)

    return k(pos_flat, ys)


def _gemm_kernel(be_ref, breal_ref, xs_ref, g_ref, u_ref, d_ref, ys_ref):
    blk = pl.program_id(0)

    @pl.when(breal_ref[blk] == 1)
    def _():
        x = xs_ref[...]
        g = g_ref[0]
        u = u_ref[0]
        d = d_ref[0]
        a = lax.dot_general(x, g, (((1,), (1,)), ((), ())),
                            preferred_element_type=jnp.float32)
        b = lax.dot_general(x, u, (((1,), (1,)), ((), ())),
                            preferred_element_type=jnp.float32)
        h = a * jax.nn.sigmoid(a) * b
        ys_ref[...] = lax.dot_general(h, d, (((1,), (1,)), ((), ())),
                                      preferred_element_type=jnp.float32)


def _grouped_gemm(xs, gate_w, up_w, down_w, block_expert, block_real):
    grid_spec = pltpu.PrefetchScalarGridSpec(
        num_scalar_prefetch=2,
        grid=(NB,),
        in_specs=[
            pl.BlockSpec((BLK, H), lambda b, be, br: (b, 0)),
            pl.BlockSpec((1, I_MOE, H), lambda b, be, br: (be[b], 0, 0)),
            pl.BlockSpec((1, I_MOE, H), lambda b, be, br: (be[b], 0, 0)),
            pl.BlockSpec((1, H, I_MOE), lambda b, be, br: (be[b], 0, 0)),
        ],
        out_specs=pl.BlockSpec((BLK, H), lambda b, be, br: (b, 0)),
    )
    return pl.pallas_call(
        _gemm_kernel,
        grid_spec=grid_spec,
        out_shape=jax.ShapeDtypeStruct((NPAD, H), jnp.float32),
    )(block_expert, block_real, xs, gate_w, up_w, down_w)


TT = 256
IT = 512


def _shared_kernel(x_ref, g_ref, u_ref, d_ref, o_ref):
    i = pl.program_id(1)
    x = x_ref[...]
    a = lax.dot_general(x, g_ref[...], (((1,), (1,)), ((), ())),
                        preferred_element_type=jnp.float32)
    b = lax.dot_general(x, u_ref[...], (((1,), (1,)), ((), ())),
                        preferred_element_type=jnp.float32)
    h = a * jax.nn.sigmoid(a) * b
    y = lax.dot_general(h, d_ref[...], (((1,), (1,)), ((), ())),
                        preferred_element_type=jnp.float32)

    @pl.when(i == 0)
    def _():
        o_ref[...] = y

    @pl.when(i > 0)
    def _():
        o_ref[...] = o_ref[...] + y


def _shared_mlp(flat, gw, uw, dw):
    return pl.pallas_call(
        _shared_kernel,
        grid=(T // TT, I_SH // IT),
        in_specs=[
            pl.BlockSpec((TT, H), lambda t, i: (t, 0)),
            pl.BlockSpec((IT, H), lambda t, i: (i, 0)),
            pl.BlockSpec((IT, H), lambda t, i: (i, 0)),
            pl.BlockSpec((H, IT), lambda t, i: (0, i)),
        ],
        out_specs=pl.BlockSpec((TT, H), lambda t, i: (t, 0)),
        out_shape=jax.ShapeDtypeStruct((T, H), jnp.float32),
    )(flat, gw, uw, dw)


CT = 64


def _combine_kernel(ysg_ref, tw_ref, sh_ref, o_ref):
    acc = sh_ref[...]
    tw = tw_ref[...]
    for k in range(TOPK):
        w = jnp.broadcast_to(tw[:, k:k + 1], (CT, H))
        acc = acc + w * ysg_ref[:, k, :]
    o_ref[...] = acc


def _combine(ysg3, tw, shared):
    return pl.pallas_call(
        _combine_kernel,
        grid=(T // CT,),
        in_specs=[
            pl.BlockSpec((CT, TOPK, H), lambda t: (t, 0, 0)),
            pl.BlockSpec((CT, TOPK), lambda t: (t, 0)),
            pl.BlockSpec((CT, H), lambda t: (t, 0)),
        ],
        out_specs=pl.BlockSpec((CT, H), lambda t: (t, 0)),
        out_shape=jax.ShapeDtypeStruct((T, H), jnp.float32),
    )(ysg3, tw, shared)


def kernel(hidden_states, router_weight, gate_w, up_w, down_w,
           sh_gate_w, sh_up_w, sh_down_w):
    orig = hidden_states.shape
    flat = hidden_states.reshape(T, H)

    tw, pos, be8, breal8 = _router(flat, router_weight)
    block_expert = be8[0, :NB]
    block_real = breal8[0, :NB]

    pos_flat = pos.reshape(NPAIR)
    init_tok = (jnp.arange(NPAD, dtype=jnp.int32) &
                jnp.int32(T - 1))
    xs = _sc_build_and_gather(pos_flat, init_tok, flat)

    shared = _shared_mlp(flat, sh_gate_w, sh_up_w, sh_down_w)

    ys = _grouped_gemm(xs, gate_w, up_w, down_w, block_expert, block_real)

    ysg3 = _sc_gather_pairs(pos_flat, ys).reshape(T, TOPK, H)

    out = _combine(ysg3, tw, shared)
    return out.reshape(orig)
